# baseline (speedup 1.0000x reference)
import jax
import jax.numpy as jnp
from jax import lax
from jax.experimental import pallas as pl
from jax.experimental.pallas import tpu as pltpu
from jax.experimental.pallas import tpu_sc as plsc

_MIN_Y = 0.2
_MAX_Y = 0.8
_MIN_MM = 0.1
_MAX_MM = 0.2
_PAD_VALUE = -80.0
_MAXY = _MAX_Y - _MAX_MM

_B, _H, _W = 64, 256, 2048
_NW = 32
_SPW = _B // _NW
_RPW = _SPW * _H
_CH = 8
_NCH = _RPW // _CH
_CPS = _H // _CH
_NBUF = 6
_LOOK = 3


def _mask_params(b, h):
    key = jax.random.key(42)
    k1, k2, k3 = jax.random.split(key, 3)
    coin = jax.random.uniform(k1, (b,), dtype=jnp.float32)
    start_f = jax.random.uniform(k2, (b,), dtype=jnp.float32, minval=_MIN_Y, maxval=_MAXY)
    width_f = jax.random.uniform(k3, (b,), dtype=jnp.float32, minval=_MIN_MM, maxval=_MAX_MM)
    start = jnp.floor(start_f * h).astype(jnp.int32)
    width = jnp.floor(width_f * h).astype(jnp.int32)
    width = jnp.where(coin <= 1.0, width, 0)
    return start, start + width


def _sc_body(x_hbm, params_hbm, out_hbm, se_v, buf_v, pad_v, isem, osem):
    wid = lax.axis_index("s") * 2 + lax.axis_index("c")
    base = wid * _RPW

    pltpu.sync_copy(params_hbm.at[pl.ds(wid, 1)], se_v)
    pv = se_v[0, :]
    s0, e0, s1, e1 = pv[0], pv[1], pv[2], pv[3]

    def _pad_row(r, _):
        def _col(j, _):
            pad_v[r, pl.ds(j * 16, 16)] = jnp.full((16,), _PAD_VALUE, jnp.float32)
            return 0

        lax.fori_loop(0, _W // 16, _col, 0, unroll=8)
        return 0

    lax.fori_loop(0, _CH, _pad_row, 0)

    def _chunk_info(c):
        r0 = (c % _CPS) * _CH
        in_second = c >= _CPS
        s = jnp.where(in_second, s1, s0)
        e = jnp.where(in_second, e1, e0)
        lo = jnp.clip(s - r0, 0, _CH)
        hi = jnp.clip(e - r0, 0, _CH)
        fullmask = (lo == 0) & (hi == _CH)
        need_read = jnp.logical_not(fullmask)
        return base + c * _CH, lo, hi, need_read

    def _issue_in(c, slot):
        g0, _, _, need_read = _chunk_info(c)

        @pl.when(need_read)
        def _():
            pltpu.async_copy(x_hbm.at[pl.ds(g0, _CH)], buf_v.at[slot], isem)

    def _wait_in(c, slot):
        _, _, _, need_read = _chunk_info(c)

        @pl.when(need_read)
        def _():
            pltpu.make_async_copy(
                x_hbm.at[pl.ds(0, _CH)], buf_v.at[slot], isem
            ).wait()

    def _wait_out(slot):
        pltpu.make_async_copy(
            buf_v.at[slot], out_hbm.at[pl.ds(0, _CH)], osem
        ).wait()

    for c in range(_LOOK):
        _issue_in(c, c % _NBUF)

    def _step(c, k):
        nxt = c + _LOOK
        slot_n = (k + _LOOK) % _NBUF

        @pl.when(nxt < _NCH)
        def _():
            @pl.when(nxt >= _NBUF)
            def _():
                _wait_out(slot_n)

            _issue_in(nxt, slot_n)

        _wait_in(c, k)

        g0, lo, hi, need_read = _chunk_info(c)

        @pl.when(need_read)
        def _():
            def _fill_row(r, _):
                def _col(j, _):
                    buf_v[k, r, pl.ds(j * 16, 16)] = jnp.full(
                        (16,), _PAD_VALUE, jnp.float32
                    )
                    return 0

                lax.fori_loop(0, _W // 16, _col, 0, unroll=8)
                return 0

            lax.fori_loop(lo, hi, _fill_row, 0)
            pltpu.async_copy(buf_v.at[k], out_hbm.at[pl.ds(g0, _CH)], osem)

        @pl.when(jnp.logical_not(need_read))
        def _():
            pltpu.async_copy(pad_v, out_hbm.at[pl.ds(g0, _CH)], osem)

    def _group(g, _):
        for k in range(_NBUF):
            _step(g * _NBUF + k, k)
        return 0

    ngroups = _NCH // _NBUF
    lax.fori_loop(0, ngroups, _group, 0)
    for c in range(ngroups * _NBUF, _NCH):
        _step(c, c % _NBUF)

    for c in range(_NCH - _NBUF, _NCH):
        _wait_out(c % _NBUF)


def kernel(x):
    b, c, h, w = x.shape
    start, end = _mask_params(b, h)
    se = jnp.stack([start, end], axis=1).reshape(_NW, 2 * _SPW)
    params = jnp.zeros((_NW, 16), jnp.int32).at[:, : 2 * _SPW].set(se)
    x2 = x.reshape(b * h, w)
    mesh = plsc.VectorSubcoreMesh(core_axis_name="c", subcore_axis_name="s")
    f = pl.kernel(
        _sc_body,
        out_type=jax.ShapeDtypeStruct((b * h, w), jnp.float32),
        mesh=mesh,
        scratch_types=[
            pltpu.VMEM((1, 16), jnp.int32),
            pltpu.VMEM((_NBUF, _CH, _W), jnp.float32),
            pltpu.VMEM((_CH, _W), jnp.float32),
            pltpu.SemaphoreType.DMA,
            pltpu.SemaphoreType.DMA,
        ],
    )
    out = f(x2, params)
    return out.reshape(b, c, h, w)


# speedup vs baseline: 1.2596x; 1.2596x over previous
import jax
import jax.numpy as jnp
from jax import lax
from jax.experimental import pallas as pl
from jax.experimental.pallas import tpu as pltpu

_MIN_Y = 0.2
_MAX_Y = 0.8
_MIN_MM = 0.1
_MAX_MM = 0.2
_PAD_VALUE = -80.0
_MAXY = _MAX_Y - _MAX_MM

_B, _H, _W = 64, 256, 2048
_NBUF = 8
_LOOK = 4


def _mask_params(b, h):
    key = jax.random.key(42)
    k1, k2, k3 = jax.random.split(key, 3)
    coin = jax.random.uniform(k1, (b,), dtype=jnp.float32)
    start_f = jax.random.uniform(k2, (b,), dtype=jnp.float32, minval=_MIN_Y, maxval=_MAXY)
    width_f = jax.random.uniform(k3, (b,), dtype=jnp.float32, minval=_MIN_MM, maxval=_MAX_MM)
    start = jnp.floor(start_f * h).astype(jnp.int32)
    width = jnp.floor(width_f * h).astype(jnp.int32)
    width = jnp.where(coin <= 1.0, width, 0)
    return start, start + width


def _body(start_ref, end_ref, x_hbm, out_hbm, buf, isem, osem):
    def _issue_in(i, k):
        pltpu.async_copy(x_hbm.at[pl.ds(i * _H, _H)], buf.at[k], isem)

    def _wait_in(k):
        pltpu.make_async_copy(x_hbm.at[pl.ds(0, _H)], buf.at[k], isem).wait()

    def _issue_out(i, k):
        pltpu.async_copy(buf.at[k], out_hbm.at[pl.ds(i * _H, _H)], osem)

    def _wait_out(k):
        pltpu.make_async_copy(buf.at[k], out_hbm.at[pl.ds(0, _H)], osem).wait()

    for i in range(_LOOK):
        _issue_in(i, i % _NBUF)

    for i in range(_B):
        k = i % _NBUF
        nxt = i + _LOOK
        if nxt < _B:
            kn = nxt % _NBUF
            if nxt >= _NBUF:
                _wait_out(kn)
            _issue_in(nxt, kn)

        _wait_in(k)

        s = start_ref[i]
        e = end_ref[i]
        s8 = (s + 7) & ~7
        e8 = e & ~7

        def _blk(bidx, _, k=k, s8=s8):
            off = pl.multiple_of(s8 + bidx * 8, 8)
            buf[k, pl.ds(off, 8), :] = jnp.full((8, _W), _PAD_VALUE, jnp.float32)
            return 0

        lax.fori_loop(0, jnp.maximum((e8 - s8) >> 3, 0), _blk, 0)

        def _row(r, _, k=k):
            buf[k, pl.ds(r, 1), :] = jnp.full((1, _W), _PAD_VALUE, jnp.float32)
            return 0

        lax.fori_loop(s, jnp.minimum(s8, e), _row, 0)
        lax.fori_loop(jnp.maximum(e8, s), e, _row, 0)

        _issue_out(i, k)

    for i in range(_B - _NBUF, _B):
        _wait_out(i % _NBUF)


def kernel(x):
    b, c, h, w = x.shape
    start, end = _mask_params(b, h)
    x2 = x.reshape(b * h, w)
    out = pl.pallas_call(
        _body,
        in_specs=[
            pl.BlockSpec(memory_space=pltpu.MemorySpace.SMEM),
            pl.BlockSpec(memory_space=pltpu.MemorySpace.SMEM),
            pl.BlockSpec(memory_space=pltpu.MemorySpace.HBM),
        ],
        out_specs=pl.BlockSpec(memory_space=pltpu.MemorySpace.HBM),
        out_shape=jax.ShapeDtypeStruct((b * h, w), jnp.float32),
        scratch_shapes=[
            pltpu.VMEM((_NBUF, _H, _W), jnp.float32),
            pltpu.SemaphoreType.DMA,
            pltpu.SemaphoreType.DMA,
        ],
    )(start, end, x2)
    return out.reshape(b, c, h, w)


# speedup vs baseline: 1.3183x; 1.0466x over previous
import jax
import jax.numpy as jnp
from jax import lax
from jax.experimental import pallas as pl
from jax.experimental.pallas import tpu as pltpu

_MIN_Y = 0.2
_MAX_Y = 0.8
_MIN_MM = 0.1
_MAX_MM = 0.2
_PAD_VALUE = -80.0
_MAXY = _MAX_Y - _MAX_MM

_B, _H, _W = 64, 256, 2048
_NBUF = 8
_LOOK = 4


def _mask_params(b, h):
    key = jax.random.key(42)
    k1, k2, k3 = jax.random.split(key, 3)
    coin = jax.random.uniform(k1, (b,), dtype=jnp.float32)
    start_f = jax.random.uniform(k2, (b,), dtype=jnp.float32, minval=_MIN_Y, maxval=_MAXY)
    width_f = jax.random.uniform(k3, (b,), dtype=jnp.float32, minval=_MIN_MM, maxval=_MAX_MM)
    start = jnp.floor(start_f * h).astype(jnp.int32)
    width = jnp.floor(width_f * h).astype(jnp.int32)
    width = jnp.where(coin <= 1.0, width, 0)
    return start, start + width


_IN_BITS = (16, 8, 4, 2, 1)


def _body(start_ref, end_ref, x_hbm, out_hbm, buf, isem, osem):
    def _in_segments(i, k, fire):
        s = start_ref[i]
        e = end_ref[i]
        hb = (s + 7) >> 3
        tb0 = e >> 3
        tb = (_H >> 3) - tb0

        def _one(off_blk, nb, cond):
            @pl.when(cond)
            def _():
                r0 = pl.multiple_of(off_blk * 8, 8)
                cp = pltpu.make_async_copy(
                    x_hbm.at[pl.ds(i * _H + r0, nb * 8)],
                    buf.at[k].at[pl.ds(r0, nb * 8)],
                    isem,
                )
                if fire:
                    cp.start()
                else:
                    cp.wait()

        off = jnp.int32(0)
        for nb in _IN_BITS:
            cond = (hb & nb) != 0
            _one(off, nb, cond)
            off = jnp.where(cond, off + nb, off)
        off = tb0
        for nb in _IN_BITS:
            cond = (tb & nb) != 0
            _one(off, nb, cond)
            off = jnp.where(cond, off + nb, off)

    def _issue_in(i, k):
        _in_segments(i, k, True)

    def _wait_in(i, k):
        _in_segments(i, k, False)

    def _issue_out(i, k):
        pltpu.async_copy(buf.at[k], out_hbm.at[pl.ds(i * _H, _H)], osem)

    def _wait_out(k):
        pltpu.make_async_copy(buf.at[k], out_hbm.at[pl.ds(0, _H)], osem).wait()

    for i in range(_LOOK):
        _issue_in(i, i % _NBUF)

    for i in range(_B):
        k = i % _NBUF
        nxt = i + _LOOK
        if nxt < _B:
            kn = nxt % _NBUF
            if nxt >= _NBUF:
                _wait_out(kn)
            _issue_in(nxt, kn)

        _wait_in(i, k)

        s = start_ref[i]
        e = end_ref[i]
        s8 = (s + 7) & ~7
        e8 = e & ~7

        def _blk(bidx, _, k=k, s8=s8):
            off = pl.multiple_of(s8 + bidx * 8, 8)
            buf[k, pl.ds(off, 8), :] = jnp.full((8, _W), _PAD_VALUE, jnp.float32)
            return 0

        lax.fori_loop(0, jnp.maximum((e8 - s8) >> 3, 0), _blk, 0)

        def _row(r, _, k=k):
            buf[k, pl.ds(r, 1), :] = jnp.full((1, _W), _PAD_VALUE, jnp.float32)
            return 0

        lax.fori_loop(s, jnp.minimum(s8, e), _row, 0)
        lax.fori_loop(jnp.maximum(e8, s), e, _row, 0)

        _issue_out(i, k)

    for i in range(_B - _NBUF, _B):
        _wait_out(i % _NBUF)


def kernel(x):
    b, c, h, w = x.shape
    start, end = _mask_params(b, h)
    x2 = x.reshape(b * h, w)
    out = pl.pallas_call(
        _body,
        in_specs=[
            pl.BlockSpec(memory_space=pltpu.MemorySpace.SMEM),
            pl.BlockSpec(memory_space=pltpu.MemorySpace.SMEM),
            pl.BlockSpec(memory_space=pltpu.MemorySpace.HBM),
        ],
        out_specs=pl.BlockSpec(memory_space=pltpu.MemorySpace.HBM),
        out_shape=jax.ShapeDtypeStruct((b * h, w), jnp.float32),
        scratch_shapes=[
            pltpu.VMEM((_NBUF, _H, _W), jnp.float32),
            pltpu.SemaphoreType.DMA,
            pltpu.SemaphoreType.DMA,
        ],
    )(start, end, x2)
    return out.reshape(b, c, h, w)


# speedup vs baseline: 1.3327x; 1.0109x over previous
import jax
import jax.numpy as jnp
from jax import lax
from jax.experimental import pallas as pl
from jax.experimental.pallas import tpu as pltpu

_MIN_Y = 0.2
_MAX_Y = 0.8
_MIN_MM = 0.1
_MAX_MM = 0.2
_PAD_VALUE = -80.0
_MAXY = _MAX_Y - _MAX_MM

_B, _H, _W = 64, 256, 2048
_SPS = 2
_NP = _B // _SPS
_NBUF = 6
_LOOK = 3


def _mask_params(b, h):
    key = jax.random.key(42)
    k1, k2, k3 = jax.random.split(key, 3)
    coin = jax.random.uniform(k1, (b,), dtype=jnp.float32)
    start_f = jax.random.uniform(k2, (b,), dtype=jnp.float32, minval=_MIN_Y, maxval=_MAXY)
    width_f = jax.random.uniform(k3, (b,), dtype=jnp.float32, minval=_MIN_MM, maxval=_MAX_MM)
    start = jnp.floor(start_f * h).astype(jnp.int32)
    width = jnp.floor(width_f * h).astype(jnp.int32)
    width = jnp.where(coin <= 1.0, width, 0)
    return start, start + width


_IN_BITS = (16, 8, 4, 2, 1)


def _body(start_ref, end_ref, x_hbm, out_hbm, buf, isem, osem):
    def _in_segments(p, k, fire):
        for j in range(_SPS):
            i = p * _SPS + j
            s = start_ref[i]
            e = end_ref[i]
            hb = (s + 7) >> 3
            tb0 = e >> 3
            tb = (_H >> 3) - tb0

            def _one(off_blk, nb, cond, j=j, i=i):
                @pl.when(cond)
                def _():
                    r0 = pl.multiple_of(off_blk * 8, 8)
                    cp = pltpu.make_async_copy(
                        x_hbm.at[pl.ds(i * _H + r0, nb * 8)],
                        buf.at[k].at[pl.ds(j * _H + r0, nb * 8)],
                        isem,
                    )
                    if fire:
                        cp.start()
                    else:
                        cp.wait()

            off = jnp.int32(0)
            for nb in _IN_BITS:
                cond = (hb & nb) != 0
                _one(off, nb, cond)
                off = jnp.where(cond, off + nb, off)
            off = tb0
            for nb in _IN_BITS:
                cond = (tb & nb) != 0
                _one(off, nb, cond)
                off = jnp.where(cond, off + nb, off)

    def _issue_in(p, k):
        _in_segments(p, k, True)

    def _wait_in(p, k):
        _in_segments(p, k, False)

    def _issue_out(p, k):
        pltpu.async_copy(buf.at[k], out_hbm.at[pl.ds(p * _SPS * _H, _SPS * _H)], osem)

    def _wait_out(k):
        pltpu.make_async_copy(
            buf.at[k], out_hbm.at[pl.ds(0, _SPS * _H)], osem
        ).wait()

    for p in range(_LOOK):
        _issue_in(p, p % _NBUF)

    for p in range(_NP):
        k = p % _NBUF
        nxt = p + _LOOK
        if nxt < _NP:
            kn = nxt % _NBUF
            if nxt >= _NBUF:
                _wait_out(kn)
            _issue_in(nxt, kn)

        _wait_in(p, k)

        for j in range(_SPS):
            i = p * _SPS + j
            s = start_ref[i] + j * _H
            e = end_ref[i] + j * _H
            s8 = (s + 7) & ~7
            e8 = e & ~7

            def _blk(bidx, _, k=k, s8=s8):
                off = pl.multiple_of(s8 + bidx * 8, 8)
                buf[k, pl.ds(off, 8), :] = jnp.full((8, _W), _PAD_VALUE, jnp.float32)
                return 0

            lax.fori_loop(0, jnp.maximum((e8 - s8) >> 3, 0), _blk, 0)

            def _row(r, _, k=k):
                buf[k, pl.ds(r, 1), :] = jnp.full((1, _W), _PAD_VALUE, jnp.float32)
                return 0

            lax.fori_loop(s, jnp.minimum(s8, e), _row, 0)
            lax.fori_loop(jnp.maximum(e8, s), e, _row, 0)

        _issue_out(p, k)

    for p in range(_NP - _NBUF, _NP):
        _wait_out(p % _NBUF)


def kernel(x):
    b, c, h, w = x.shape
    start, end = _mask_params(b, h)
    x2 = x.reshape(b * h, w)
    out = pl.pallas_call(
        _body,
        in_specs=[
            pl.BlockSpec(memory_space=pltpu.MemorySpace.SMEM),
            pl.BlockSpec(memory_space=pltpu.MemorySpace.SMEM),
            pl.BlockSpec(memory_space=pltpu.MemorySpace.HBM),
        ],
        out_specs=pl.BlockSpec(memory_space=pltpu.MemorySpace.HBM),
        out_shape=jax.ShapeDtypeStruct((b * h, w), jnp.float32),
        scratch_shapes=[
            pltpu.VMEM((_NBUF, _SPS * _H, _W), jnp.float32),
            pltpu.SemaphoreType.DMA,
            pltpu.SemaphoreType.DMA,
        ],
    )(start, end, x2)
    return out.reshape(b, c, h, w)
